# speedup vs baseline: 1.0008x; 1.0008x over previous
import functools
import math

import jax
import jax.numpy as jnp
import numpy as np
from jax.experimental import pallas as pl
from jax.experimental.pallas import tpu as pltpu

N = 10000
E = 320000
F = 128
H = 8
D = 16
L = 4
G = 64

ROW_BLK = 1000


def _qkvs_body(h_ref, w_ref, b_ref, out_ref):
    out_ref[...] = (
        jnp.dot(h_ref[...], w_ref[...], preferred_element_type=jnp.float32)
        + b_ref[...]
    )


def _qkvs(h, W, b):
    return pl.pallas_call(
        _qkvs_body,
        grid=(N // ROW_BLK,),
        in_specs=[
            pl.BlockSpec((ROW_BLK, F), lambda i: (i, 0)),
            pl.BlockSpec((F, 4 * F), lambda i: (0, 0)),
            pl.BlockSpec((1, 4 * F), lambda i: (0, 0)),
        ],
        out_specs=pl.BlockSpec((ROW_BLK, 4 * F), lambda i: (i, 0)),
        out_shape=jax.ShapeDtypeStruct((N, 4 * F), jnp.float32),
    )(h, W, b.reshape(1, 4 * F))


def _ln_gelu_body(x_ref, g_ref, b_ref, res_ref, out_ref):
    x = x_ref[...]
    mu = jnp.mean(x, axis=-1, keepdims=True)
    var = jnp.mean((x - mu) ** 2, axis=-1, keepdims=True)
    y = (x - mu) * jax.lax.rsqrt(var + 1e-5) * g_ref[...] + b_ref[...]
    y = 0.5 * y * (1.0 + jax.lax.erf(y / math.sqrt(2.0)))
    out_ref[...] = y + res_ref[...]


def _ln_gelu_res(x, gamma, beta, res):
    return pl.pallas_call(
        _ln_gelu_body,
        grid=(N // ROW_BLK,),
        in_specs=[
            pl.BlockSpec((ROW_BLK, F), lambda i: (i, 0)),
            pl.BlockSpec((1, F), lambda i: (0, 0)),
            pl.BlockSpec((1, F), lambda i: (0, 0)),
            pl.BlockSpec((ROW_BLK, F), lambda i: (i, 0)),
        ],
        out_specs=pl.BlockSpec((ROW_BLK, F), lambda i: (i, 0)),
        out_shape=jax.ShapeDtypeStruct((N, F), jnp.float32),
    )(x, gamma.reshape(1, F), beta.reshape(1, F), res)


def kernel(x, edge_index, batch, W_in, b_in, Wq, bq, Wk, bk, Wv, bv, Ws, bs,
           gamma, beta, W1, b1, W2, b2, W3, b3):
    src = edge_index[0]
    dst = edge_index[1]
    h = x @ W_in + b_in
    for i in range(L):
        res = h
        Wcat = jnp.concatenate([Wq[i], Wk[i], Wv[i], Ws[i]], axis=1)
        bcat = jnp.concatenate([bq[i], bk[i], bv[i], bs[i]], axis=0)
        qkvs = _qkvs(h, Wcat, bcat)
        q = qkvs[:, 0:F].reshape(N, H, D)
        k = qkvs[:, F:2 * F].reshape(N, H, D)
        v = qkvs[:, 2 * F:3 * F].reshape(N, H, D)
        s = qkvs[:, 3 * F:4 * F]
        alpha = jnp.sum(q[dst] * k[src], axis=-1) / np.sqrt(D)
        amax = jax.ops.segment_max(alpha, dst, num_segments=N)
        a = jnp.exp(alpha - amax[dst])
        asum = jax.ops.segment_sum(a, dst, num_segments=N)
        a = a / (asum[dst] + 1e-16)
        agg = jax.ops.segment_sum(v[src] * a[:, :, None], dst,
                                  num_segments=N).reshape(N, H * D)
        h2 = agg + s
        h = _ln_gelu_res(h2, gamma[i], beta[i], res)
    ones = jnp.ones((N,), h.dtype)
    counts = jax.ops.segment_sum(ones, batch, num_segments=G)
    ssum = jax.ops.segment_sum(h, batch, num_segments=G)
    mean = ssum / jnp.maximum(counts, 1.0)[:, None]
    smax = jax.ops.segment_max(h, batch, num_segments=G)
    pooled = jnp.concatenate([mean, smax, ssum], axis=1)
    z = jax.nn.relu(pooled @ W1 + b1)
    z = jax.nn.relu(z @ W2 + b2)
    return z @ W3 + b3


# speedup vs baseline: 33.9968x; 33.9687x over previous
import functools
import math

import jax
import jax.numpy as jnp
import numpy as np
from jax import lax
from jax.experimental import pallas as pl
from jax.experimental.pallas import tpu as pltpu
from jax.experimental.pallas import tpu_sc as plsc

N = 10000
E = 320000
F = 128
H = 8
D = 16
L = 4
G = 64

NC = 2
NS = 16
NW = NC * NS
EPW = E // NW
GC = 80
GITERS = EPW // GC
NHALF = 5000
NACC = 5120
NPT = NACC // NS

ROW_BLK = 1000
EDGE_BLK = 2000

_SMAT = np.repeat(np.eye(H, dtype=np.float32), D, axis=0)
_SMATT = _SMAT.T.copy()

_MESH = plsc.VectorSubcoreMesh(core_axis_name="c", subcore_axis_name="s",
                               num_cores=NC, num_subcores=NS)


def _make_gather(n_tables):
    scratch = []
    for _b in range(2):
        scratch.append(pltpu.VMEM((GC,), jnp.int32))
    for _b in range(2):
        for _t in range(n_tables):
            scratch.append(pltpu.VMEM((GC, F), jnp.float32))
    nsem = 2 + 2 * n_tables * 2
    for _ in range(nsem):
        scratch.append(pltpu.SemaphoreType.DMA)

    out_type = tuple(jax.ShapeDtypeStruct((E, F), jnp.float32)
                     for _ in range(n_tables))

    @functools.partial(pl.kernel, out_type=out_type, mesh=_MESH,
                       scratch_types=scratch)
    def gather(*args):
        tables = args[:n_tables]
        idx_hbm = args[n_tables]
        outs = args[n_tables + 1: 2 * n_tables + 1]
        rest = args[2 * n_tables + 1:]
        ibufs = rest[0:2]
        rbufs = (rest[2:2 + n_tables], rest[2 + n_tables:2 + 2 * n_tables])
        sems = rest[2 + 2 * n_tables:]
        sis = sems[0:2]
        sgs = (sems[2:2 + n_tables], sems[2 + n_tables:2 + 2 * n_tables])
        sws = (sems[2 + 2 * n_tables:2 + 3 * n_tables],
               sems[2 + 3 * n_tables:2 + 4 * n_tables])

        cid = lax.axis_index("c")
        sid = lax.axis_index("s")
        wid = sid * NC + cid
        base = wid * EPW

        def start_idx(j, b):
            pltpu.async_copy(idx_hbm.at[pl.ds(base + j * GC, GC)],
                             ibufs[b], sis[b])

        def wait_idx(b):
            pltpu.make_async_copy(idx_hbm.at[pl.ds(base, GC)],
                                  ibufs[b], sis[b]).wait()

        def start_gathers(b):
            for t in range(n_tables):
                pltpu.async_copy(tables[t].at[ibufs[b]], rbufs[b][t],
                                 sgs[b][t])

        def wait_gathers(b):
            for t in range(n_tables):
                pltpu.make_async_copy(tables[t].at[ibufs[b]], rbufs[b][t],
                                      sgs[b][t]).wait()

        def wait_writebacks(b):
            for t in range(n_tables):
                pltpu.make_async_copy(rbufs[b][t],
                                      outs[t].at[pl.ds(base, GC)],
                                      sws[b][t]).wait()

        def start_writebacks(j, b):
            for t in range(n_tables):
                pltpu.async_copy(rbufs[b][t],
                                 outs[t].at[pl.ds(base + j * GC, GC)],
                                 sws[b][t])

        start_idx(0, 0)
        start_idx(1, 1)
        wait_idx(0)
        start_gathers(0)

        def chunk(j, b):
            nb = 1 - b

            @pl.when(j + 1 < GITERS)
            def _():
                wait_idx(nb)

                @pl.when(j >= 1)
                def _():
                    wait_writebacks(nb)
                start_gathers(nb)

            wait_gathers(b)

            @pl.when(j + 2 < GITERS)
            def _():
                start_idx(j + 2, b)
            start_writebacks(j, b)

        def pair(m, carry):
            chunk(2 * m, 0)
            chunk(2 * m + 1, 1)
            return carry

        lax.fori_loop(0, GITERS // 2, pair, 0)
        jlast = GITERS - 1
        wait_gathers(0)
        start_writebacks(jlast, 0)
        wait_writebacks(1)
        wait_writebacks(0)

    return gather


_GATHER2 = _make_gather(2)
_GATHER1 = _make_gather(1)


def _make_scatter(half):
    scratch = [
        pltpu.VMEM((GC,), jnp.int32),
        pltpu.VMEM((GC,), jnp.int32),
        pltpu.VMEM((GC,), jnp.int32),
        pltpu.VMEM((GC,), jnp.int32),
        pltpu.VMEM((GC, F), jnp.float32),
        pltpu.VMEM((GC, F), jnp.float32),
        pltpu.VMEM((NPT, F), jnp.float32),
        pltpu.VMEM_SHARED((NACC, F), jnp.float32),
        pltpu.SemaphoreType.DMA,
        pltpu.SemaphoreType.DMA,
        pltpu.SemaphoreType.DMA,
        pltpu.SemaphoreType.DMA,
    ]

    @functools.partial(
        pl.kernel,
        out_type=jax.ShapeDtypeStruct((NC, NACC, F), jnp.float32),
        mesh=_MESH,
        scratch_types=scratch,
    )
    def scatter(vals_hbm, idx_hbm, zrows_hbm, out_hbm,
                i0, i1, m0, m1, v0, v1, zb, acc, si0, si1, sv0, sv1):
        ibufs = (i0, i1)
        mbufs = (m0, m1)
        vbufs = (v0, v1)
        sis = (si0, si1)
        svs = (sv0, sv1)

        cid = lax.axis_index("c")
        sid = lax.axis_index("s")
        wid = sid * NC + cid
        base = wid * EPW

        pltpu.sync_copy(zrows_hbm, zb)
        pltpu.sync_copy(zb, acc.at[pl.ds(sid * NPT, NPT)])
        plsc.subcore_barrier()

        def start_loads(j, b):
            pltpu.async_copy(idx_hbm.at[pl.ds(base + j * GC, GC)],
                             ibufs[b], sis[b])
            pltpu.async_copy(vals_hbm.at[pl.ds(base + j * GC, GC)],
                             vbufs[b], svs[b])

        def wait_loads(b):
            pltpu.make_async_copy(idx_hbm.at[pl.ds(base, GC)],
                                  ibufs[b], sis[b]).wait()
            pltpu.make_async_copy(vals_hbm.at[pl.ds(base, GC)],
                                  vbufs[b], svs[b]).wait()

        start_loads(0, 0)
        start_loads(1, 1)

        def chunk(j, b):
            wait_loads(b)
            for t in range(GC // 16):
                w = ibufs[b][pl.ds(t * 16, 16)] - (half * NHALF)
                ok = (w >= 0) & (w < NHALF)
                mbufs[b][pl.ds(t * 16, 16)] = jnp.where(ok, w, NHALF)
            pltpu.sync_copy(vbufs[b], acc.at[mbufs[b]], add=True)

            @pl.when(j + 2 < GITERS)
            def _():
                start_loads(j + 2, b)

        def pair(m, carry):
            chunk(2 * m, 0)
            chunk(2 * m + 1, 1)
            return carry

        lax.fori_loop(0, GITERS // 2, pair, 0)
        chunk(GITERS - 1, 0)

        plsc.subcore_barrier()
        pltpu.sync_copy(acc.at[pl.ds(sid * NPT, NPT)], zb)
        pltpu.sync_copy(zb, out_hbm.at[cid, pl.ds(sid * NPT, NPT)])

    return scatter


_SCATTER_H = (_make_scatter(0), _make_scatter(1))


def _mm_body(x_ref, w_ref, b_ref, out_ref):
    out_ref[...] = (
        jnp.dot(x_ref[...], w_ref[...], preferred_element_type=jnp.float32)
        + b_ref[...]
    )


def _mm_bias(x, W, b):
    Wo = W.shape[1]
    return pl.pallas_call(
        _mm_body,
        grid=(N // ROW_BLK,),
        in_specs=[
            pl.BlockSpec((ROW_BLK, F), lambda i: (i, 0)),
            pl.BlockSpec((F, Wo), lambda i: (0, 0)),
            pl.BlockSpec((1, Wo), lambda i: (0, 0)),
        ],
        out_specs=pl.BlockSpec((ROW_BLK, Wo), lambda i: (i, 0)),
        out_shape=jax.ShapeDtypeStruct((N, Wo), jnp.float32),
    )(x, W, b.reshape(1, Wo))


def _proj_body(h_ref, w_ref, b_ref, smat_ref, q_ref, k_ref, v_ref, s_ref,
               kn_ref):
    hw = (jnp.dot(h_ref[...], w_ref[...], preferred_element_type=jnp.float32)
          + b_ref[...])
    k = hw[:, F:2 * F]
    q_ref[...] = hw[:, 0:F]
    k_ref[...] = k
    v_ref[...] = hw[:, 2 * F:3 * F]
    s_ref[...] = hw[:, 3 * F:4 * F]
    kn_ref[...] = jnp.sqrt(
        jnp.dot(k * k, smat_ref[...], preferred_element_type=jnp.float32))


def _proj(h, Wcat, bcat, smat):
    outs = [
        jax.ShapeDtypeStruct((N, F), jnp.float32),
        jax.ShapeDtypeStruct((N, F), jnp.float32),
        jax.ShapeDtypeStruct((N, F), jnp.float32),
        jax.ShapeDtypeStruct((N, F), jnp.float32),
        jax.ShapeDtypeStruct((N, H), jnp.float32),
    ]
    return pl.pallas_call(
        _proj_body,
        grid=(N // ROW_BLK,),
        in_specs=[
            pl.BlockSpec((ROW_BLK, F), lambda i: (i, 0)),
            pl.BlockSpec((F, 4 * F), lambda i: (0, 0)),
            pl.BlockSpec((1, 4 * F), lambda i: (0, 0)),
            pl.BlockSpec((F, H), lambda i: (0, 0)),
        ],
        out_specs=[
            pl.BlockSpec((ROW_BLK, F), lambda i: (i, 0)),
            pl.BlockSpec((ROW_BLK, F), lambda i: (i, 0)),
            pl.BlockSpec((ROW_BLK, F), lambda i: (i, 0)),
            pl.BlockSpec((ROW_BLK, F), lambda i: (i, 0)),
            pl.BlockSpec((ROW_BLK, H), lambda i: (i, 0)),
        ],
        out_shape=outs,
    )(h, Wcat, bcat.reshape(1, 4 * F), smat)


def _edge_body(qd_ref, k_ref, v_ref, kmax_ref, smat_ref, smatt_ref,
               pe_ref, pv_ref):
    qd = qd_ref[...]
    ks = k_ref[...]
    vs = v_ref[...]
    scale = 1.0 / math.sqrt(D)
    alpha = jnp.dot(qd * ks, smat_ref[...],
                    preferred_element_type=jnp.float32) * scale
    qn = jnp.sqrt(jnp.dot(qd * qd, smat_ref[...],
                          preferred_element_type=jnp.float32))
    c = qn * (kmax_ref[...] * scale)
    p = jnp.exp(alpha - c)
    pexp = jnp.dot(p, smatt_ref[...], preferred_element_type=jnp.float32)
    pe_ref[...] = pexp
    pv_ref[...] = pexp * vs


def _edge(qd_rows, k_rows, v_rows, kmax, smat, smatt):
    outs = [
        jax.ShapeDtypeStruct((E, F), jnp.float32),
        jax.ShapeDtypeStruct((E, F), jnp.float32),
    ]
    return pl.pallas_call(
        _edge_body,
        grid=(E // EDGE_BLK,),
        in_specs=[
            pl.BlockSpec((EDGE_BLK, F), lambda i: (i, 0)),
            pl.BlockSpec((EDGE_BLK, F), lambda i: (i, 0)),
            pl.BlockSpec((EDGE_BLK, F), lambda i: (i, 0)),
            pl.BlockSpec((1, H), lambda i: (0, 0)),
            pl.BlockSpec((F, H), lambda i: (0, 0)),
            pl.BlockSpec((H, F), lambda i: (0, 0)),
        ],
        out_specs=[
            pl.BlockSpec((EDGE_BLK, F), lambda i: (i, 0)),
            pl.BlockSpec((EDGE_BLK, F), lambda i: (i, 0)),
        ],
        out_shape=outs,
    )(qd_rows, k_rows, v_rows, kmax, smat, smatt)


def _ln_body(agg2_ref, ps2_ref, s_ref, res_ref, g_ref, b_ref, out_ref):
    agg = agg2_ref[0] + agg2_ref[1]
    psum = ps2_ref[0] + ps2_ref[1]
    x = agg / (psum + 1e-16) + s_ref[...]
    mu = jnp.mean(x, axis=-1, keepdims=True)
    var = jnp.mean((x - mu) ** 2, axis=-1, keepdims=True)
    y = (x - mu) * jax.lax.rsqrt(var + 1e-5) * g_ref[...] + b_ref[...]
    y = 0.5 * y * (1.0 + jax.lax.erf(y / math.sqrt(2.0)))
    out_ref[...] = y + res_ref[...]


def _ln_gelu_res_half(agg2, ps2, s, res, gamma, beta, half):
    off = half * (NHALF // ROW_BLK)
    return pl.pallas_call(
        _ln_body,
        grid=(NHALF // ROW_BLK,),
        in_specs=[
            pl.BlockSpec((NC, ROW_BLK, F), lambda i: (0, i, 0)),
            pl.BlockSpec((NC, ROW_BLK, F), lambda i: (0, i, 0)),
            pl.BlockSpec((ROW_BLK, F), lambda i: (i + off, 0)),
            pl.BlockSpec((ROW_BLK, F), lambda i: (i + off, 0)),
            pl.BlockSpec((1, F), lambda i: (0, 0)),
            pl.BlockSpec((1, F), lambda i: (0, 0)),
        ],
        out_specs=pl.BlockSpec((ROW_BLK, F), lambda i: (i, 0)),
        out_shape=jax.ShapeDtypeStruct((NHALF, F), jnp.float32),
    )(agg2, ps2, s, res, gamma.reshape(1, F), beta.reshape(1, F))


def _pool_body(h_ref, batch_ref, ssum_ref, smax_ref, cnt_ref):
    i = pl.program_id(0)

    @pl.when(i == 0)
    def _():
        ssum_ref[...] = jnp.zeros((G, F), jnp.float32)
        smax_ref[...] = jnp.full((G, F), -jnp.inf, jnp.float32)
        cnt_ref[...] = jnp.zeros((G, F), jnp.float32)

    h = h_ref[...]
    b = batch_ref[...]
    iota_g = lax.broadcasted_iota(jnp.int32, (ROW_BLK, G), 1)
    oh = (b == iota_g).astype(jnp.float32)
    dn = (((0,), (0,)), ((), ()))
    ssum_ref[...] += lax.dot_general(oh, h, dn,
                                     preferred_element_type=jnp.float32)
    cnt_ref[...] += lax.dot_general(
        oh, jnp.ones((ROW_BLK, F), jnp.float32), dn,
        preferred_element_type=jnp.float32)
    neg = jnp.float32(-jnp.inf)
    rows = [jnp.max(jnp.where(b == g, h, neg), axis=0, keepdims=True)
            for g in range(G)]
    smax_ref[...] = jnp.maximum(smax_ref[...], jnp.concatenate(rows, axis=0))


def _pool(h, batch2d):
    outs = [
        jax.ShapeDtypeStruct((G, F), jnp.float32),
        jax.ShapeDtypeStruct((G, F), jnp.float32),
        jax.ShapeDtypeStruct((G, F), jnp.float32),
    ]
    return pl.pallas_call(
        _pool_body,
        grid=(N // ROW_BLK,),
        in_specs=[
            pl.BlockSpec((ROW_BLK, F), lambda i: (i, 0)),
            pl.BlockSpec((ROW_BLK, 1), lambda i: (i, 0)),
        ],
        out_specs=[
            pl.BlockSpec((G, F), lambda i: (0, 0)),
            pl.BlockSpec((G, F), lambda i: (0, 0)),
            pl.BlockSpec((G, F), lambda i: (0, 0)),
        ],
        out_shape=outs,
    )(h, batch2d)


def _mlp_body(ssum_ref, smax_ref, cnt_ref, w1a_ref, w1b_ref, w1c_ref, b1_ref,
              w2_ref, b2_ref, w3_ref, b3_ref, out_ref):
    ssum = ssum_ref[...]
    cnt = jnp.maximum(cnt_ref[...], 1.0)
    mean = ssum / cnt
    z = (jnp.dot(mean, w1a_ref[...], preferred_element_type=jnp.float32)
         + jnp.dot(smax_ref[...], w1b_ref[...],
                   preferred_element_type=jnp.float32)
         + jnp.dot(ssum, w1c_ref[...], preferred_element_type=jnp.float32)
         + b1_ref[...])
    z = jnp.maximum(z, 0.0)
    z = jnp.dot(z, w2_ref[...], preferred_element_type=jnp.float32) + b2_ref[...]
    z = jnp.maximum(z, 0.0)
    out_ref[...] = (jnp.dot(z, w3_ref[...], preferred_element_type=jnp.float32)
                    + b3_ref[...])


def _mlp(ssum, smax, cnt, W1, b1, W2, b2, W3, b3):
    return pl.pallas_call(
        _mlp_body,
        in_specs=[pl.BlockSpec(a.shape, lambda: tuple(0 for _ in a.shape))
                  for a in (ssum, smax, cnt, W1[0:F], W1[F:2 * F],
                            W1[2 * F:3 * F], b1.reshape(1, 2 * F), W2,
                            b2.reshape(1, F), W3, b3.reshape(1, 1))],
        out_specs=pl.BlockSpec((G, 1), lambda: (0, 0)),
        out_shape=jax.ShapeDtypeStruct((G, 1), jnp.float32),
    )(ssum, smax, cnt, W1[0:F], W1[F:2 * F], W1[2 * F:3 * F],
      b1.reshape(1, 2 * F), W2, b2.reshape(1, F), W3, b3.reshape(1, 1))


def kernel(x, edge_index, batch, W_in, b_in, Wq, bq, Wk, bk, Wv, bv, Ws, bs,
           gamma, beta, W1, b1, W2, b2, W3, b3):
    src = edge_index[0]
    dst = edge_index[1]
    smat = jnp.asarray(_SMAT)
    smatt = jnp.asarray(_SMATT)
    zrows = jnp.zeros((NPT, F), jnp.float32)

    h = _mm_bias(x, W_in, b_in)
    for i in range(L):
        res = h
        Wcat = jnp.concatenate([Wq[i], Wk[i], Wv[i], Ws[i]], axis=1)
        bcat = jnp.concatenate([bq[i], bk[i], bv[i], bs[i]], axis=0)
        q, k, v, s, kn = _proj(h, Wcat, bcat, smat)
        kmax = jnp.max(kn, axis=0).reshape(1, H)
        k_rows, v_rows = _GATHER2(k, v, src)
        (qd_rows,) = _GATHER1(q, dst)
        pe, pv = _edge(qd_rows, k_rows, v_rows, kmax, smat, smatt)
        halves = []
        for half in range(2):
            ps2 = _SCATTER_H[half](pe, dst, zrows)
            agg2 = _SCATTER_H[half](pv, dst, zrows)
            halves.append(_ln_gelu_res_half(agg2, ps2, s, res,
                                            gamma[i], beta[i], half))
        h = jnp.concatenate(halves, axis=0)

    ssum, smax, cnt = _pool(h, batch.reshape(N, 1))
    return _mlp(ssum, smax, cnt, W1, b1, W2, b2, W3, b3)
